# baseline (speedup 1.0000x reference)
import functools

import jax
import jax.numpy as jnp
from jax import lax
from jax.experimental import pallas as pl
from jax.experimental.pallas import tpu as pltpu
from jax.experimental.pallas import tpu_sc as plsc

N_NODES = 50000
K_MAX = 8
INPUT_BITS = 512
BATCH = 128
N_OUT = 10

NW = 32
NPAD = 51200
NT = NPAD // NW
BLK = 512
NBLK = NPAD // BLK
NG = 16
NQ = NPAD // 4
NBR = 98
ZROW = 50176


def _prep_body(states_ref, x_ref, wcol_ref, psel_ref, qlo_ref, qhi_ref,
               lut_ref, wplo_ref, wphi_ref, adjt_ref, maskt_ref,
               tbl_ref, lutw_ref, idxt_ref):
    i = pl.program_id(0)
    base = i * BLK
    wcol = wcol_ref[...]
    jcol = base + lax.broadcasted_iota(jnp.int32, (BLK, BLK), 1)
    m = (wcol == jcol).astype(jnp.float32)
    s = jnp.dot(x_ref[...], m, preferred_element_type=jnp.float32)
    pert = (states_ref[...] ^ s.astype(jnp.int32)).astype(jnp.float32)
    pk = jnp.dot(psel_ref[...], pert, preferred_element_type=jnp.float32)
    lo = jnp.dot(pk, qlo_ref[...], preferred_element_type=jnp.float32)
    hi = jnp.dot(pk, qhi_ref[...], preferred_element_type=jnp.float32)
    tbl_ref[...] = lo.astype(jnp.int32) | (hi.astype(jnp.int32) << 16)
    lut_f = lut_ref[...].astype(jnp.float32)
    llo = jnp.dot(lut_f, wplo_ref[...], preferred_element_type=jnp.float32)
    lhi = jnp.dot(lut_f, wphi_ref[...], preferred_element_type=jnp.float32)
    lutw_ref[...] = llo.astype(jnp.int32) | (lhi.astype(jnp.int32) << 16)
    mk = jnp.transpose(maskt_ref[...])
    adt = jnp.transpose(adjt_ref[...])
    idxt_ref[...] = jnp.clip(adt * mk + ZROW * (1 - mk), 0, ZROW)


def _prep(states_p, x_f, w_col, psel, qlo, qhi, lut_p, wp_lo, wp_hi,
          adj_t, mask_t):
    return pl.pallas_call(
        _prep_body,
        grid=(NBLK,),
        in_specs=[
            pl.BlockSpec((BATCH, BLK), lambda i: (0, jnp.minimum(i, NBR - 1))),
            pl.BlockSpec((BATCH, INPUT_BITS), lambda i: (0, 0)),
            pl.BlockSpec((INPUT_BITS, 1), lambda i: (0, 0)),
            pl.BlockSpec((NG, BATCH), lambda i: (0, 0)),
            pl.BlockSpec((BLK, BLK // 4), lambda i: (0, 0)),
            pl.BlockSpec((BLK, BLK // 4), lambda i: (0, 0)),
            pl.BlockSpec((BLK, 256), lambda i: (jnp.minimum(i, NBR - 1), 0)),
            pl.BlockSpec((256, 8), lambda i: (0, 0)),
            pl.BlockSpec((256, 8), lambda i: (0, 0)),
            pl.BlockSpec((BLK, K_MAX), lambda i: (jnp.minimum(i, NBR - 1), 0)),
            pl.BlockSpec((BLK, K_MAX), lambda i: (jnp.minimum(i, NBR - 1), 0)),
        ],
        out_specs=[
            pl.BlockSpec((NG, BLK // 4), lambda i: (0, i)),
            pl.BlockSpec((BLK, 8), lambda i: (i, 0)),
            pl.BlockSpec((K_MAX, BLK), lambda i: (0, i)),
        ],
        out_shape=[
            jax.ShapeDtypeStruct((NG, NQ), jnp.int32),
            jax.ShapeDtypeStruct((NPAD, 8), jnp.int32),
            jax.ShapeDtypeStruct((K_MAX, NPAD), jnp.int32),
        ],
    )(states_p, x_f, w_col, psel, qlo, qhi, lut_p, wp_lo, wp_hi, adj_t, mask_t)


@functools.partial(
    pl.kernel,
    out_type=jax.ShapeDtypeStruct((NG * NPAD,), jnp.int32),
    mesh=plsc.VectorSubcoreMesh(core_axis_name="c", subcore_axis_name="s"),
    compiler_params=pltpu.CompilerParams(needs_layout_passes=False),
    scratch_types=[
        pltpu.VMEM((NQ,), jnp.int32),
        pltpu.VMEM((K_MAX * NT,), jnp.int32),
        pltpu.VMEM((NT * 8,), jnp.int32),
        pltpu.VMEM((NT,), jnp.int32),
    ],
)
def _sc_update(tblf_hbm, idxt_hbm, lutwf_hbm, out_hbm,
               tbl_v, idxt_v, lutw_v, row_v):
    wid = lax.axis_index("c") * 16 + lax.axis_index("s")
    nbase = wid * NT
    for k in range(K_MAX):
        pltpu.sync_copy(idxt_hbm.at[pl.ds(k * NPAD + nbase, NT)],
                        idxt_v.at[pl.ds(k * NT, NT)])
    pltpu.sync_copy(lutwf_hbm.at[pl.ds(nbase * 8, NT * 8)], lutw_v)
    iota16 = lax.iota(jnp.int32, 16)
    i8v = iota16 * 8

    def pass_body(p, carry):
        pltpu.sync_copy(tblf_hbm.at[pl.ds(p * NQ, NQ)], tbl_v)
        tbl_v[pl.ds(ZROW // 4, 16)] = iota16 * 0

        def grp_body(g, carry2):
            base = g * 16
            nl8 = base * 8 + i8v
            ws = []
            for k in range(K_MAX):
                a = idxt_v[pl.ds(k * NT + base, 16)]
                w = plsc.load_gather(tbl_v, [a >> 2])
                ws.append(w >> ((a & 3) << 3))
            accw = None
            for b in range(8):
                addr = ((ws[0] >> b) & 1) << 7
                for k in range(1, K_MAX):
                    addr = addr | (((ws[k] >> b) & 1) << (7 - k))
                word = plsc.load_gather(lutw_v, [nl8 + (addr >> 5)])
                bit = (word >> (addr & 31)) & 1
                bw = bit << b
                accw = bw if accw is None else accw | bw
            row_v[pl.ds(base, 16)] = accw
            return carry2

        lax.fori_loop(0, NT // 16, grp_body, 0)
        pltpu.sync_copy(row_v, out_hbm.at[pl.ds(p * NPAD + nbase, NT)])
        return carry

    lax.fori_loop(0, NG, pass_body, 0)


def _readout_body(opk_ref, w_ref, b_ref, e_ref, out_ref):
    i = pl.program_id(0)
    vf = jnp.transpose(opk_ref[...]).astype(jnp.float32)
    acc = jnp.dot(vf, e_ref[...],
                  preferred_element_type=jnp.float32).astype(jnp.int32)
    bl = lax.broadcasted_iota(jnp.int32, (BLK, BATCH), 1)
    bits = ((acc >> (bl & 7)) & 1).astype(jnp.float32)
    part = jnp.dot(w_ref[...], bits, preferred_element_type=jnp.float32)

    @pl.when(i == 0)
    def _init():
        out_ref[...] = jnp.zeros_like(out_ref)

    out_ref[...] += part

    @pl.when(i == NBLK - 1)
    def _fin():
        z = out_ref[...] + b_ref[...]
        out_ref[...] = 1.0 / (1.0 + jnp.exp(-z))


def _readout(opk, w_pad, b2d, emat):
    return pl.pallas_call(
        _readout_body,
        grid=(NBLK,),
        in_specs=[
            pl.BlockSpec((NG, BLK), lambda i: (0, i)),
            pl.BlockSpec((16, BLK), lambda i: (0, i)),
            pl.BlockSpec((16, BATCH), lambda i: (0, 0)),
            pl.BlockSpec((NG, BATCH), lambda i: (0, 0)),
        ],
        out_specs=pl.BlockSpec((16, BATCH), lambda i: (0, 0)),
        out_shape=jax.ShapeDtypeStruct((16, BATCH), jnp.float32),
    )(opk, w_pad, b2d, emat)


def kernel(x, states, adj_list, adj_list_mask, lut, powers_of_2, w_in, W_out, b_out):
    del powers_of_2
    x_f = x.astype(jnp.float32)
    w_col = w_in.reshape(INPUT_BITS, 1).astype(jnp.int32)
    brow = jnp.arange(BATCH, dtype=jnp.int32)
    psel = ((brow[None, :] >> 3 == jnp.arange(NG, dtype=jnp.int32)[:, None])
            .astype(jnp.float32) * (2.0 ** (brow % 8).astype(jnp.float32))[None, :])
    jrow = jnp.arange(BLK, dtype=jnp.int32)
    qsel = (jrow[:, None] >> 2 == jnp.arange(BLK // 4, dtype=jnp.int32)[None, :])
    qpw = (2.0 ** (8.0 * (jrow % 4).astype(jnp.float32)))
    qlo = (qsel & (jrow[:, None] % 4 < 2)).astype(jnp.float32) * qpw[:, None]
    qhi = ((qsel & (jrow[:, None] % 4 >= 2)).astype(jnp.float32)
           * (2.0 ** (8.0 * (jrow % 4 - 2).astype(jnp.float32)))[:, None])
    cols = jnp.arange(256, dtype=jnp.int32)
    sel = (cols[:, None] // 32 == jnp.arange(8, dtype=jnp.int32)[None, :])
    pw = (2.0 ** (cols % 16).astype(jnp.float32))[:, None]
    wp_lo = (sel & (cols[:, None] % 32 < 16)).astype(jnp.float32) * pw
    wp_hi = (sel & (cols[:, None] % 32 >= 16)).astype(jnp.float32) * pw
    w_pad = jnp.pad(W_out, ((0, 16 - N_OUT), (0, NPAD - N_NODES)))
    b2d = jnp.broadcast_to(
        jnp.pad(b_out, (0, 16 - N_OUT)).reshape(16, 1), (16, BATCH))

    tbl, lutw, idx_t = _prep(states, x_f, w_col, psel, qlo, qhi, lut,
                             wp_lo, wp_hi, adj_list, adj_list_mask)
    opk = _sc_update(tbl.reshape(-1), idx_t.reshape(-1), lutw.reshape(-1))
    emat = (brow[None, :] >> 3 ==
            jnp.arange(NG, dtype=jnp.int32)[:, None]).astype(jnp.float32)
    sig = _readout(opk.reshape(NG, NPAD), w_pad, b2d, emat)
    return jnp.transpose(sig[:N_OUT, :])


# speedup vs baseline: 1.0886x; 1.0886x over previous
import functools

import jax
import jax.numpy as jnp
from jax import lax
from jax.experimental import pallas as pl
from jax.experimental.pallas import tpu as pltpu
from jax.experimental.pallas import tpu_sc as plsc

N_NODES = 50000
K_MAX = 8
INPUT_BITS = 512
BATCH = 128
N_OUT = 10

NW = 32
NPAD = 51200
NT = NPAD // NW
BLK = 512
NBLK = NPAD // BLK
NG = 16
NQ = NPAD // 4
NBR = 98
ZROW = 50176


def _prep_body(states_ref, x_ref, wcol_ref, psel_ref, qlo_ref, qhi_ref,
               lut_ref, wplo_ref, wphi_ref, adjt_ref, maskt_ref,
               tbl_ref, lutw_ref, idxt_ref):
    i = pl.program_id(0)
    base = i * BLK
    wcol = wcol_ref[...]
    jcol = base + lax.broadcasted_iota(jnp.int32, (BLK, BLK), 1)
    m = (wcol == jcol).astype(jnp.float32)
    s = jnp.dot(x_ref[...], m, preferred_element_type=jnp.float32)
    pert = (states_ref[...] ^ s.astype(jnp.int32)).astype(jnp.float32)
    pk = jnp.dot(psel_ref[...], pert, preferred_element_type=jnp.float32)
    lo = jnp.dot(pk, qlo_ref[...], preferred_element_type=jnp.float32)
    hi = jnp.dot(pk, qhi_ref[...], preferred_element_type=jnp.float32)
    tbl_ref[...] = lo.astype(jnp.int32) | (hi.astype(jnp.int32) << 16)
    lut_f = lut_ref[...].astype(jnp.float32)
    llo = jnp.dot(lut_f, wplo_ref[...], preferred_element_type=jnp.float32)
    lhi = jnp.dot(lut_f, wphi_ref[...], preferred_element_type=jnp.float32)
    lutw_ref[...] = llo.astype(jnp.int32) | (lhi.astype(jnp.int32) << 16)
    mk = maskt_ref[...]
    idxt_ref[...] = jnp.clip(adjt_ref[...] * mk + ZROW * (1 - mk), 0, ZROW)


def _prep(states_p, x_f, w_col, psel, qlo, qhi, lut_p, wp_lo, wp_hi,
          adj_t, mask_t):
    return pl.pallas_call(
        _prep_body,
        grid=(NBLK,),
        in_specs=[
            pl.BlockSpec((BATCH, BLK), lambda i: (0, jnp.minimum(i, NBR - 1))),
            pl.BlockSpec((BATCH, INPUT_BITS), lambda i: (0, 0)),
            pl.BlockSpec((INPUT_BITS, 1), lambda i: (0, 0)),
            pl.BlockSpec((NG, BATCH), lambda i: (0, 0)),
            pl.BlockSpec((BLK, BLK // 4), lambda i: (0, 0)),
            pl.BlockSpec((BLK, BLK // 4), lambda i: (0, 0)),
            pl.BlockSpec((BLK, 256), lambda i: (jnp.minimum(i, NBR - 1), 0)),
            pl.BlockSpec((256, 8), lambda i: (0, 0)),
            pl.BlockSpec((256, 8), lambda i: (0, 0)),
            pl.BlockSpec((K_MAX, BLK), lambda i: (0, jnp.minimum(i, NBR - 1))),
            pl.BlockSpec((K_MAX, BLK), lambda i: (0, jnp.minimum(i, NBR - 1))),
        ],
        out_specs=[
            pl.BlockSpec((NG, BLK // 4), lambda i: (0, i)),
            pl.BlockSpec((BLK, 8), lambda i: (i, 0)),
            pl.BlockSpec((K_MAX, BLK), lambda i: (0, i)),
        ],
        out_shape=[
            jax.ShapeDtypeStruct((NG, NQ), jnp.int32),
            jax.ShapeDtypeStruct((NPAD, 8), jnp.int32),
            jax.ShapeDtypeStruct((K_MAX, NPAD), jnp.int32),
        ],
    )(states_p, x_f, w_col, psel, qlo, qhi, lut_p, wp_lo, wp_hi, adj_t, mask_t)


@functools.partial(
    pl.kernel,
    out_type=jax.ShapeDtypeStruct((NG * NPAD,), jnp.int32),
    mesh=plsc.VectorSubcoreMesh(core_axis_name="c", subcore_axis_name="s"),
    compiler_params=pltpu.CompilerParams(needs_layout_passes=False),
    scratch_types=[
        pltpu.VMEM((NQ,), jnp.int32),
        pltpu.VMEM((K_MAX * NT,), jnp.int32),
        pltpu.VMEM((NT * 8,), jnp.int32),
        pltpu.VMEM((NT,), jnp.int32),
    ],
)
def _sc_update(tblf_hbm, idxt_hbm, lutwf_hbm, out_hbm,
               tbl_v, idxt_v, lutw_v, row_v):
    wid = lax.axis_index("c") * 16 + lax.axis_index("s")
    nbase = wid * NT
    for k in range(K_MAX):
        pltpu.sync_copy(idxt_hbm.at[pl.ds(k * NPAD + nbase, NT)],
                        idxt_v.at[pl.ds(k * NT, NT)])
    pltpu.sync_copy(lutwf_hbm.at[pl.ds(nbase * 8, NT * 8)], lutw_v)
    iota16 = lax.iota(jnp.int32, 16)
    i8v = iota16 * 8

    def pass_body(p, carry):
        pltpu.sync_copy(tblf_hbm.at[pl.ds(p * NQ, NQ)], tbl_v)
        tbl_v[pl.ds(ZROW // 4, 16)] = iota16 * 0

        def grp_body(g, carry2):
            base = g * 16
            nl8 = base * 8 + i8v
            ws = []
            for k in range(K_MAX):
                a = idxt_v[pl.ds(k * NT + base, 16)]
                w = plsc.load_gather(tbl_v, [a >> 2])
                ws.append(w >> ((a & 3) << 3))
            accw = None
            for b in range(8):
                addr = None
                for k in range(K_MAX):
                    pos = 7 - k
                    if b >= pos:
                        t = (ws[k] >> (b - pos)) & (1 << pos)
                    else:
                        t = (ws[k] << (pos - b)) & (1 << pos)
                    addr = t if addr is None else addr | t
                word = plsc.load_gather(lutw_v, [nl8 + (addr >> 5)])
                bit = (word >> (addr & 31)) & 1
                bw = bit << b
                accw = bw if accw is None else accw | bw
            row_v[pl.ds(base, 16)] = accw
            return carry2

        lax.fori_loop(0, NT // 16, grp_body, 0)
        pltpu.sync_copy(row_v, out_hbm.at[pl.ds(p * NPAD + nbase, NT)])
        return carry

    lax.fori_loop(0, NG, pass_body, 0)


def _readout_body(opk_ref, w_ref, b_ref, e_ref, out_ref):
    i = pl.program_id(0)
    vf = jnp.transpose(opk_ref[...]).astype(jnp.float32)
    acc = jnp.dot(vf, e_ref[...],
                  preferred_element_type=jnp.float32).astype(jnp.int32)
    bl = lax.broadcasted_iota(jnp.int32, (BLK, BATCH), 1)
    bits = ((acc >> (bl & 7)) & 1).astype(jnp.float32)
    part = jnp.dot(w_ref[...], bits, preferred_element_type=jnp.float32)

    @pl.when(i == 0)
    def _init():
        out_ref[...] = jnp.zeros_like(out_ref)

    out_ref[...] += part

    @pl.when(i == NBLK - 1)
    def _fin():
        z = out_ref[...] + b_ref[...]
        out_ref[...] = 1.0 / (1.0 + jnp.exp(-z))


def _readout(opk, w_pad, b2d, emat):
    return pl.pallas_call(
        _readout_body,
        grid=(NBLK,),
        in_specs=[
            pl.BlockSpec((NG, BLK), lambda i: (0, i)),
            pl.BlockSpec((16, BLK), lambda i: (0, i)),
            pl.BlockSpec((16, BATCH), lambda i: (0, 0)),
            pl.BlockSpec((NG, BATCH), lambda i: (0, 0)),
        ],
        out_specs=pl.BlockSpec((16, BATCH), lambda i: (0, 0)),
        out_shape=jax.ShapeDtypeStruct((16, BATCH), jnp.float32),
    )(opk, w_pad, b2d, emat)


def kernel(x, states, adj_list, adj_list_mask, lut, powers_of_2, w_in, W_out, b_out):
    del powers_of_2
    adj_t = jnp.transpose(adj_list)
    mask_t = jnp.transpose(adj_list_mask)
    x_f = x.astype(jnp.float32)
    w_col = w_in.reshape(INPUT_BITS, 1).astype(jnp.int32)
    brow = jnp.arange(BATCH, dtype=jnp.int32)
    psel = ((brow[None, :] >> 3 == jnp.arange(NG, dtype=jnp.int32)[:, None])
            .astype(jnp.float32) * (2.0 ** (brow % 8).astype(jnp.float32))[None, :])
    jrow = jnp.arange(BLK, dtype=jnp.int32)
    qsel = (jrow[:, None] >> 2 == jnp.arange(BLK // 4, dtype=jnp.int32)[None, :])
    qpw = (2.0 ** (8.0 * (jrow % 4).astype(jnp.float32)))
    qlo = (qsel & (jrow[:, None] % 4 < 2)).astype(jnp.float32) * qpw[:, None]
    qhi = ((qsel & (jrow[:, None] % 4 >= 2)).astype(jnp.float32)
           * (2.0 ** (8.0 * (jrow % 4 - 2).astype(jnp.float32)))[:, None])
    cols = jnp.arange(256, dtype=jnp.int32)
    sel = (cols[:, None] // 32 == jnp.arange(8, dtype=jnp.int32)[None, :])
    pw = (2.0 ** (cols % 16).astype(jnp.float32))[:, None]
    wp_lo = (sel & (cols[:, None] % 32 < 16)).astype(jnp.float32) * pw
    wp_hi = (sel & (cols[:, None] % 32 >= 16)).astype(jnp.float32) * pw
    w_pad = jnp.pad(W_out, ((0, 16 - N_OUT), (0, NPAD - N_NODES)))
    b2d = jnp.broadcast_to(
        jnp.pad(b_out, (0, 16 - N_OUT)).reshape(16, 1), (16, BATCH))

    tbl, lutw, idx_t = _prep(states, x_f, w_col, psel, qlo, qhi, lut,
                             wp_lo, wp_hi, adj_t, mask_t)
    opk = _sc_update(tbl.reshape(-1), idx_t.reshape(-1), lutw.reshape(-1))
    emat = (brow[None, :] >> 3 ==
            jnp.arange(NG, dtype=jnp.int32)[:, None]).astype(jnp.float32)
    sig = _readout(opk.reshape(NG, NPAD), w_pad, b2d, emat)
    return jnp.transpose(sig[:N_OUT, :])
